# baseline (speedup 1.0000x reference)
import functools

import jax
import jax.numpy as jnp
from jax import lax
from jax.experimental import pallas as pl
from jax.experimental.pallas import tpu as pltpu
from jax.experimental.pallas import tpu_sc as plsc

_NUM_CORES = 2
_NUM_SUBCORES = 16
_NW = _NUM_CORES * _NUM_SUBCORES

_CHUNK = 32
_NBUF = 3


@functools.partial(jax.jit, static_argnums=(2, 3))
def _sc_gather(table, idx, n_per_w, d):
  n_chunks = n_per_w // _CHUNK
  mesh = plsc.VectorSubcoreMesh(
      core_axis_name="c", subcore_axis_name="s",
      num_cores=_NUM_CORES, num_subcores=_NUM_SUBCORES)

  @functools.partial(
      pl.kernel,
      out_type=jax.ShapeDtypeStruct((idx.shape[0], d), jnp.float32),
      mesh=mesh,
      scratch_types=(
          pltpu.VMEM((n_per_w,), jnp.int32),
          [pltpu.VMEM((_CHUNK, d), jnp.float32) for _ in range(_NBUF)],
          [pltpu.SemaphoreType.DMA for _ in range(_NBUF)],
          [pltpu.SemaphoreType.DMA for _ in range(_NBUF)],
      ),
  )
  def body(table_hbm, idx_hbm, out_hbm, idx_v, bufs, gsems, wsems):
    wid = lax.axis_index("s") * _NUM_CORES + lax.axis_index("c")
    base = wid * n_per_w
    pltpu.sync_copy(idx_hbm.at[pl.ds(base, n_per_w)], idx_v)

    def gather(c):
      return pltpu.async_copy(
          table_hbm.at[idx_v.at[pl.ds(c * _CHUNK, _CHUNK)]],
          bufs[c % _NBUF],
          gsems[c % _NBUF],
      )

    gd = [gather(c) for c in range(min(_NBUF, n_chunks))]
    for c in range(n_chunks):
      gd[c % _NBUF].wait()
      wd = pltpu.async_copy(
          bufs[c % _NBUF],
          out_hbm.at[pl.ds(base + c * _CHUNK, _CHUNK)],
          wsems[c % _NBUF],
      )
      wd.wait()
      if c + _NBUF < n_chunks:
        gd[c % _NBUF] = gather(c + _NBUF)

  return body(table, idx)


def kernel(input_ids, table):
  b, s = input_ids.shape
  v, d = table.shape
  n = b * s
  flat = input_ids.reshape(n).astype(jnp.int32)
  out = _sc_gather(table, flat, n // _NW, d)
  return out.reshape(b, s, d)
